# baseline (speedup 1.0000x reference)
import functools

import jax
import jax.numpy as jnp
from jax.experimental import pallas as pl
import jax.experimental.pallas.tpu as pltpu

N = 8192
I_BLK = 256
J_BLK = 1024
BIG = 3.0e38


def _nn_kernel(a_ref, b_ref, esdf_ref, idx_ref, acc_val, acc_idx):
    i = pl.program_id(0)
    j = pl.program_id(1)

    @pl.when(j == 0)
    def _init():
        acc_val[...] = jnp.full((I_BLK, J_BLK), BIG, jnp.float32)
        acc_idx[...] = jnp.zeros((I_BLK, J_BLK), jnp.int32)

    a = a_ref[...]
    b = b_ref[...]
    ax = a[:, 0]
    ay = a[:, 1]
    bx = b[0, :]
    by = b[1, :]
    sq_i = ax * ax + ay * ay
    sq_j = bx * bx + by * by
    m = jnp.dot(a, b, preferred_element_type=jnp.float32)
    t = sq_i[:, None] + sq_j[None, :]
    d2 = t - 2.0 * m
    d2 = jnp.maximum(d2, jnp.float32(1e-12))

    jj = jax.lax.broadcasted_iota(jnp.int32, (I_BLK, J_BLK), 1) + j * J_BLK

    def _update(vals):
        mask = vals < acc_val[...]
        acc_idx[...] = jnp.where(mask, jj, acc_idx[...])
        acc_val[...] = jnp.where(mask, vals, acc_val[...])

    diag_hit = (i // (J_BLK // I_BLK)) == j

    @pl.when(diag_hit)
    def _with_diag():
        ii = jax.lax.broadcasted_iota(jnp.int32, (I_BLK, J_BLK), 0) + i * I_BLK
        _update(jnp.where(ii == jj, BIG, d2))

    @pl.when(jnp.logical_not(diag_hit))
    def _no_diag():
        _update(d2)

    @pl.when(j == (N // J_BLK) - 1)
    def _finish():
        av = acc_val[...]
        ai = acc_idx[...]
        rmin = jnp.min(av, axis=1)
        cand = jnp.where(av == rmin[:, None], ai, jnp.int32(2**30))
        ridx = jnp.min(cand, axis=1)
        esdf_ref[pl.ds(i * I_BLK, I_BLK)] = jnp.sqrt(rmin)
        idx_ref[pl.ds(i * I_BLK, I_BLK)] = ridx


@functools.partial(jax.jit, static_argnames=())
def _nn_argmin(point_cloud):
    pcT = point_cloud.T
    esdf, idx = pl.pallas_call(
        _nn_kernel,
        grid=(N // I_BLK, N // J_BLK),
        in_specs=[
            pl.BlockSpec((I_BLK, 2), lambda i, j: (i, 0)),
            pl.BlockSpec((2, J_BLK), lambda i, j: (0, j)),
        ],
        out_specs=[
            pl.BlockSpec((N,), lambda i, j: (0,)),
            pl.BlockSpec((N,), lambda i, j: (0,)),
        ],
        out_shape=[
            jax.ShapeDtypeStruct((N,), jnp.float32),
            jax.ShapeDtypeStruct((N,), jnp.int32),
        ],
        scratch_shapes=[
            pltpu.VMEM((I_BLK, J_BLK), jnp.float32),
            pltpu.VMEM((I_BLK, J_BLK), jnp.int32),
        ],
    )(point_cloud, pcT)
    return esdf, idx


def kernel(point_cloud):
    esdf, idx = _nn_argmin(point_cloud)
    nearest = point_cloud[idx]
    direction = point_cloud - nearest
    norm = jnp.linalg.norm(direction, axis=1, keepdims=True)
    gradients = direction / (norm + 1e-8)
    gx = gradients[:, 0]
    gy = gradients[:, 1]
    mu = jnp.stack([gx, -gx, gy, -gy], axis=0)
    lam = jnp.stack([gx, gy, esdf / 10.0], axis=0)
    return (mu, lam)


# speedup vs baseline: 1.0591x; 1.0591x over previous
import functools

import jax
import jax.numpy as jnp
from jax.experimental import pallas as pl
import jax.experimental.pallas.tpu as pltpu

N = 8192
I_BLK = 256
J_BLK = 1024
BIG = 3.0e38


LANES = 128
CHUNKS = J_BLK // LANES


def _nn_kernel(a_ref, b_ref, esdf_ref, idx_ref, acc_val, acc_idx):
    i = pl.program_id(0)
    j = pl.program_id(1)

    @pl.when(j == 0)
    def _init():
        acc_val[...] = jnp.full((I_BLK, LANES), BIG, jnp.float32)
        acc_idx[...] = jnp.zeros((I_BLK, LANES), jnp.int32)

    a = a_ref[...]
    b = b_ref[...]
    ax = a[:, 0]
    ay = a[:, 1]
    bx = b[0, :]
    by = b[1, :]
    sq_i = ax * ax + ay * ay
    sq_j = bx * bx + by * by
    m2 = 2.0 * jnp.dot(a, b, preferred_element_type=jnp.float32)
    sqi_b = sq_i[:, None]

    def _scan_tile(mask_diag):
        tile_v = None
        tile_c = None
        if mask_diag:
            ii = jax.lax.broadcasted_iota(jnp.int32, (I_BLK, J_BLK), 0) + i * I_BLK
            jj = jax.lax.broadcasted_iota(jnp.int32, (I_BLK, J_BLK), 1) + j * J_BLK
            self_mask = ii == jj
        for c in range(CHUNKS):
            sl = slice(c * LANES, (c + 1) * LANES)
            t = sqi_b + sq_j[None, sl]
            d2 = jnp.maximum(t - m2[:, sl], jnp.float32(1e-12))
            if mask_diag:
                d2 = jnp.where(self_mask[:, sl], BIG, d2)
            if c == 0:
                tile_v = d2
                tile_c = jnp.zeros((I_BLK, LANES), jnp.int32)
            else:
                better = d2 < tile_v
                tile_c = jnp.where(better, jnp.int32(c), tile_c)
                tile_v = jnp.minimum(tile_v, d2)
        lane = jax.lax.broadcasted_iota(jnp.int32, (I_BLK, LANES), 1)
        tile_idx = (tile_c * LANES + lane) + j * J_BLK
        better = tile_v < acc_val[...]
        acc_idx[...] = jnp.where(better, tile_idx, acc_idx[...])
        acc_val[...] = jnp.minimum(acc_val[...], tile_v)

    diag_hit = (i // (J_BLK // I_BLK)) == j

    @pl.when(diag_hit)
    def _with_diag():
        _scan_tile(True)

    @pl.when(jnp.logical_not(diag_hit))
    def _no_diag():
        _scan_tile(False)

    @pl.when(j == (N // J_BLK) - 1)
    def _finish():
        av = acc_val[...]
        ai = acc_idx[...]
        rmin = jnp.min(av, axis=1)
        cand = jnp.where(av == rmin[:, None], ai, jnp.int32(2**30))
        ridx = jnp.min(cand, axis=1)
        esdf_ref[pl.ds(i * I_BLK, I_BLK)] = jnp.sqrt(rmin)
        idx_ref[pl.ds(i * I_BLK, I_BLK)] = ridx


@functools.partial(jax.jit, static_argnames=())
def _nn_argmin(point_cloud):
    pcT = point_cloud.T
    esdf, idx = pl.pallas_call(
        _nn_kernel,
        grid=(N // I_BLK, N // J_BLK),
        in_specs=[
            pl.BlockSpec((I_BLK, 2), lambda i, j: (i, 0)),
            pl.BlockSpec((2, J_BLK), lambda i, j: (0, j)),
        ],
        out_specs=[
            pl.BlockSpec((N,), lambda i, j: (0,)),
            pl.BlockSpec((N,), lambda i, j: (0,)),
        ],
        out_shape=[
            jax.ShapeDtypeStruct((N,), jnp.float32),
            jax.ShapeDtypeStruct((N,), jnp.int32),
        ],
        scratch_shapes=[
            pltpu.VMEM((I_BLK, LANES), jnp.float32),
            pltpu.VMEM((I_BLK, LANES), jnp.int32),
        ],
    )(point_cloud, pcT)
    return esdf, idx


def kernel(point_cloud):
    esdf, idx = _nn_argmin(point_cloud)
    nearest = point_cloud[idx]
    direction = point_cloud - nearest
    norm = jnp.linalg.norm(direction, axis=1, keepdims=True)
    gradients = direction / (norm + 1e-8)
    gx = gradients[:, 0]
    gy = gradients[:, 1]
    mu = jnp.stack([gx, -gx, gy, -gy], axis=0)
    lam = jnp.stack([gx, gy, esdf / 10.0], axis=0)
    return (mu, lam)


# speedup vs baseline: 1.3189x; 1.2453x over previous
import functools

import jax
import jax.numpy as jnp
from jax.experimental import pallas as pl
import jax.experimental.pallas.tpu as pltpu

N = 8192
I_BLK = 256
J_BLK = 2048
LANES = 128
CHUNKS = J_BLK // LANES
BIG = 3.0e38


def _prep_kernel(pc_ref, pcT_ref, sqi_ref, sqj_ref):
    xc = pc_ref[:, 0:1]
    yc = pc_ref[:, 1:2]
    sq_col = xc * xc + yc * yc
    sqi_ref[...] = jnp.broadcast_to(sq_col, (N, LANES))
    xr = pcT_ref[0:1, :]
    yr = pcT_ref[1:2, :]
    sq_row = xr * xr + yr * yr
    sqj_ref[...] = jnp.broadcast_to(sq_row, (8, N))


def _nn_kernel(a_ref, b_ref, sqi_ref, sqj_ref, esdf_ref, idx_ref,
               acc_val, acc_idx):
    i = pl.program_id(0)
    j = pl.program_id(1)

    @pl.when(j == 0)
    def _init():
        acc_val[...] = jnp.full((I_BLK, LANES), BIG, jnp.float32)
        acc_idx[...] = jnp.zeros((I_BLK, LANES), jnp.int32)

    m2 = 2.0 * jnp.dot(a_ref[...], b_ref[...],
                       preferred_element_type=jnp.float32)
    sqi = sqi_ref[...]
    sqj = sqj_ref[...]

    def _scan_tile(mask_diag):
        tile_v = None
        tile_c = None
        if mask_diag:
            ii = jax.lax.broadcasted_iota(jnp.int32, (I_BLK, J_BLK), 0) + i * I_BLK
            jj = jax.lax.broadcasted_iota(jnp.int32, (I_BLK, J_BLK), 1) + j * J_BLK
            self_mask = ii == jj
        for c in range(CHUNKS):
            sl = slice(c * LANES, (c + 1) * LANES)
            sqj_c = jnp.broadcast_to(
                sqj[:, sl].reshape(1, 8, LANES),
                (I_BLK // 8, 8, LANES)).reshape(I_BLK, LANES)
            t = sqi + sqj_c
            d2 = jnp.maximum(t - m2[:, sl], jnp.float32(1e-12))
            if mask_diag:
                d2 = jnp.where(self_mask[:, sl], BIG, d2)
            if c == 0:
                tile_v = d2
                tile_c = jnp.zeros((I_BLK, LANES), jnp.int32)
            else:
                better = d2 < tile_v
                tile_c = jnp.where(better, jnp.int32(c), tile_c)
                tile_v = jnp.minimum(tile_v, d2)
        lane = jax.lax.broadcasted_iota(jnp.int32, (I_BLK, LANES), 1)
        tile_idx = (tile_c * LANES + lane) + j * J_BLK
        better = tile_v < acc_val[...]
        acc_idx[...] = jnp.where(better, tile_idx, acc_idx[...])
        acc_val[...] = jnp.minimum(acc_val[...], tile_v)

    diag_hit = (i // (J_BLK // I_BLK)) == j

    @pl.when(diag_hit)
    def _with_diag():
        _scan_tile(True)

    @pl.when(jnp.logical_not(diag_hit))
    def _no_diag():
        _scan_tile(False)

    @pl.when(j == (N // J_BLK) - 1)
    def _finish():
        av = acc_val[...]
        ai = acc_idx[...]
        rmin = jnp.min(av, axis=1)
        cand = jnp.where(av == rmin[:, None], ai, jnp.int32(2**30))
        ridx = jnp.min(cand, axis=1)
        esdf_ref[pl.ds(i * I_BLK, I_BLK)] = jnp.sqrt(rmin)
        idx_ref[pl.ds(i * I_BLK, I_BLK)] = ridx


@jax.jit
def _nn_argmin(point_cloud):
    pcT = point_cloud.T
    sqi_b, sqj_b = pl.pallas_call(
        _prep_kernel,
        out_shape=[
            jax.ShapeDtypeStruct((N, LANES), jnp.float32),
            jax.ShapeDtypeStruct((8, N), jnp.float32),
        ],
    )(point_cloud, pcT)
    esdf, idx = pl.pallas_call(
        _nn_kernel,
        grid=(N // I_BLK, N // J_BLK),
        in_specs=[
            pl.BlockSpec((I_BLK, 2), lambda i, j: (i, 0)),
            pl.BlockSpec((2, J_BLK), lambda i, j: (0, j)),
            pl.BlockSpec((I_BLK, LANES), lambda i, j: (i, 0)),
            pl.BlockSpec((8, J_BLK), lambda i, j: (0, j)),
        ],
        out_specs=[
            pl.BlockSpec((N,), lambda i, j: (0,)),
            pl.BlockSpec((N,), lambda i, j: (0,)),
        ],
        out_shape=[
            jax.ShapeDtypeStruct((N,), jnp.float32),
            jax.ShapeDtypeStruct((N,), jnp.int32),
        ],
        scratch_shapes=[
            pltpu.VMEM((I_BLK, LANES), jnp.float32),
            pltpu.VMEM((I_BLK, LANES), jnp.int32),
        ],
    )(point_cloud, pcT, sqi_b, sqj_b)
    return esdf, idx


def kernel(point_cloud):
    esdf, idx = _nn_argmin(point_cloud)
    nearest = point_cloud[idx]
    direction = point_cloud - nearest
    norm = jnp.linalg.norm(direction, axis=1, keepdims=True)
    gradients = direction / (norm + 1e-8)
    gx = gradients[:, 0]
    gy = gradients[:, 1]
    mu = jnp.stack([gx, -gx, gy, -gy], axis=0)
    lam = jnp.stack([gx, gy, esdf / 10.0], axis=0)
    return (mu, lam)


# speedup vs baseline: 1.3656x; 1.0354x over previous
import jax
import jax.numpy as jnp
from jax.experimental import pallas as pl
import jax.experimental.pallas.tpu as pltpu

N = 8192
I_BLK = 256
J_BLK = 2048
LANES = 128
CHUNKS = J_BLK // LANES
RG = I_BLK // 8
BIG = 3.0e38


def _prep_kernel(pc_ref, pcT_ref, sqi_ref, sqj_ref):
    xc = pc_ref[:, 0:1]
    yc = pc_ref[:, 1:2]
    sq_col = xc * xc + yc * yc
    sqi_ref[...] = jnp.broadcast_to(sq_col, (N, LANES))
    xr = pcT_ref[0:1, :]
    yr = pcT_ref[1:2, :]
    sq_row = xr * xr + yr * yr
    sqj_ref[...] = jnp.broadcast_to(sq_row, (8, N))


def _nn_kernel(a_ref, b_ref, sqi_ref, sqj_ref, esdf_ref, idx_ref,
               m2_ref, acc_val, acc_idx):
    i = pl.program_id(0)
    j = pl.program_id(1)

    @pl.when(j == 0)
    def _init():
        acc_val[...] = jnp.full((I_BLK, LANES), BIG, jnp.float32)
        acc_idx[...] = jnp.zeros((I_BLK, LANES), jnp.int32)

    m2_ref[...] = jnp.dot(a_ref[...] * 2.0, b_ref[...],
                          preferred_element_type=jnp.float32)

    @pl.when((i // (J_BLK // I_BLK)) == j)
    def _patch_diag():
        off = i * I_BLK - j * J_BLK
        slab = m2_ref[:, pl.ds(off, I_BLK)]
        rr = jax.lax.broadcasted_iota(jnp.int32, (I_BLK, I_BLK), 0)
        cc = jax.lax.broadcasted_iota(jnp.int32, (I_BLK, I_BLK), 1)
        m2_ref[:, pl.ds(off, I_BLK)] = jnp.where(rr == cc, -BIG, slab)

    sqi = sqi_ref[...].reshape(RG, 8, LANES)
    tile_v = None
    tile_c = None
    for c in range(CHUNKS):
        sl = slice(c * LANES, (c + 1) * LANES)
        sqj_c = sqj_ref[:, sl].reshape(1, 8, LANES)
        t = sqi + sqj_c
        d2 = jnp.maximum(t - m2_ref[:, sl].reshape(RG, 8, LANES),
                         jnp.float32(1e-12))
        if c == 0:
            tile_v = d2
            tile_c = jnp.zeros((RG, 8, LANES), jnp.int32)
        else:
            better = d2 < tile_v
            tile_c = jnp.where(better, jnp.int32(c), tile_c)
            tile_v = jnp.minimum(tile_v, d2)

    tile_v = tile_v.reshape(I_BLK, LANES)
    tile_c = tile_c.reshape(I_BLK, LANES)
    lane = jax.lax.broadcasted_iota(jnp.int32, (I_BLK, LANES), 1)
    tile_idx = (tile_c * LANES) + (lane + j * J_BLK)
    better = tile_v < acc_val[...]
    acc_idx[...] = jnp.where(better, tile_idx, acc_idx[...])
    acc_val[...] = jnp.minimum(acc_val[...], tile_v)

    @pl.when(j == (N // J_BLK) - 1)
    def _finish():
        av = acc_val[...]
        ai = acc_idx[...]
        rmin = jnp.min(av, axis=1)
        cand = jnp.where(av == rmin[:, None], ai, jnp.int32(2**30))
        ridx = jnp.min(cand, axis=1)
        esdf_ref[pl.ds(i * I_BLK, I_BLK)] = jnp.sqrt(rmin)
        idx_ref[pl.ds(i * I_BLK, I_BLK)] = ridx


@jax.jit
def _nn_argmin(point_cloud):
    pcT = point_cloud.T
    sqi_b, sqj_b = pl.pallas_call(
        _prep_kernel,
        out_shape=[
            jax.ShapeDtypeStruct((N, LANES), jnp.float32),
            jax.ShapeDtypeStruct((8, N), jnp.float32),
        ],
    )(point_cloud, pcT)
    esdf, idx = pl.pallas_call(
        _nn_kernel,
        grid=(N // I_BLK, N // J_BLK),
        in_specs=[
            pl.BlockSpec((I_BLK, 2), lambda i, j: (i, 0)),
            pl.BlockSpec((2, J_BLK), lambda i, j: (0, j)),
            pl.BlockSpec((I_BLK, LANES), lambda i, j: (i, 0)),
            pl.BlockSpec((8, J_BLK), lambda i, j: (0, j)),
        ],
        out_specs=[
            pl.BlockSpec((N,), lambda i, j: (0,)),
            pl.BlockSpec((N,), lambda i, j: (0,)),
        ],
        out_shape=[
            jax.ShapeDtypeStruct((N,), jnp.float32),
            jax.ShapeDtypeStruct((N,), jnp.int32),
        ],
        scratch_shapes=[
            pltpu.VMEM((I_BLK, J_BLK), jnp.float32),
            pltpu.VMEM((I_BLK, LANES), jnp.float32),
            pltpu.VMEM((I_BLK, LANES), jnp.int32),
        ],
    )(point_cloud, pcT, sqi_b, sqj_b)
    return esdf, idx


def kernel(point_cloud):
    esdf, idx = _nn_argmin(point_cloud)
    nearest = point_cloud[idx]
    direction = point_cloud - nearest
    norm = jnp.linalg.norm(direction, axis=1, keepdims=True)
    gradients = direction / (norm + 1e-8)
    gx = gradients[:, 0]
    gy = gradients[:, 1]
    mu = jnp.stack([gx, -gx, gy, -gy], axis=0)
    lam = jnp.stack([gx, gy, esdf / 10.0], axis=0)
    return (mu, lam)
